# baseline (speedup 1.0000x reference)
import jax
import jax.numpy as jnp
from jax import lax
from jax.experimental import pallas as pl
from jax.experimental.pallas import tpu as pltpu
from jax.experimental.pallas import tpu_sc as plsc

N = 10000
E = 320000
D = 128
G = 64

NC = 2
NS = 16
DH = D

K = 64
CHUNKS = 160
W_EDGES = CHUNKS * K
E_PAD = NC * NS * W_EDGES
N_ACC = 10240
ROWS_PER_TILE = N_ACC // NS
DUMMY_DST = N + 8

NBLK = 1000
NSTEPS = N // NBLK


def _sc_body(xh, srcp, dstp, zrows, zcnt, ones_hbm,
             agg_out, cnt_out,
             src_all, dst_all, rows_a, rows_b, ones_v, acc_sh, cnt_sh,
             gsem0, gsem1, ssem0, ssem1):
    c = lax.axis_index("c")
    s = lax.axis_index("s")
    rows = (rows_a, rows_b)
    gsem = (gsem0, gsem1)
    ssem = (ssem0, ssem1)

    wid = s * NC + c
    pltpu.sync_copy(srcp.at[pl.ds(wid * CHUNKS, CHUNKS)], src_all)
    pltpu.sync_copy(dstp.at[pl.ds(wid * CHUNKS, CHUNKS)], dst_all)
    pltpu.sync_copy(ones_hbm, ones_v)

    base = s * ROWS_PER_TILE
    pltpu.sync_copy(zrows, acc_sh.at[pl.ds(base, ROWS_PER_TILE)])
    pltpu.sync_copy(zcnt, cnt_sh.at[pl.ds(base, ROWS_PER_TILE)])
    plsc.subcore_barrier()

    x_half = xh

    def start_gather(j, b):
        pltpu.async_copy(x_half.at[src_all.at[j]], rows[b], gsem[b])

    def wait_gather(j, b):
        pltpu.make_async_copy(x_half.at[src_all.at[j]], rows[b],
                              gsem[b]).wait()

    def start_scatter(j, b):
        pltpu.async_copy(rows[b], acc_sh.at[dst_all.at[j]], ssem[b],
                         add=True)

    def wait_scatter(j, b):
        pltpu.make_async_copy(rows[b], acc_sh.at[dst_all.at[j]],
                              ssem[b]).wait()

    def count(j):
        pltpu.sync_copy(ones_v, cnt_sh.at[dst_all.at[j]], add=True)

    start_gather(0, 0)
    wait_gather(0, 0)
    start_scatter(0, 0)
    start_gather(1, 1)

    def pair(p, carry):
        for b in range(2):
            j = 2 * p + 1 + b
            wait_scatter(j - 1, b)
            start_gather(j + 1, b)
            wait_gather(j, 1 - b)
            start_scatter(j, 1 - b)

            count(j - 1)
        return carry

    lax.fori_loop(0, CHUNKS // 2 - 1, pair, 0)
    j_last = CHUNKS - 1
    wait_scatter(j_last - 1, 0)
    wait_gather(j_last, 1)
    start_scatter(j_last, 1)
    wait_scatter(j_last, 1)

    count(j_last - 1)
    count(j_last)

    plsc.subcore_barrier()

    pltpu.sync_copy(acc_sh.at[pl.ds(base, ROWS_PER_TILE)],
                    agg_out.at[c].at[pl.ds(base, ROWS_PER_TILE)])

    pltpu.sync_copy(cnt_sh.at[pl.ds(base, ROWS_PER_TILE)],
                    cnt_out.at[c].at[pl.ds(base, ROWS_PER_TILE)])


@jax.jit
def _sc_aggregate(xh, srcp, dstp):
    zrows = jnp.zeros((ROWS_PER_TILE, DH), jnp.float32)
    zcnt = jnp.zeros((ROWS_PER_TILE,), jnp.float32)
    ones = jnp.ones((K,), jnp.float32)
    mesh = plsc.VectorSubcoreMesh(core_axis_name="c", subcore_axis_name="s")
    return pl.kernel(
        _sc_body,
        out_type=(
            jax.ShapeDtypeStruct((NC, N_ACC, DH), jnp.float32),
            jax.ShapeDtypeStruct((NC, N_ACC), jnp.float32),
        ),
        mesh=mesh,
        compiler_params=pltpu.CompilerParams(use_tc_tiling_on_sc=False),
        scratch_types=[
            pltpu.VMEM((CHUNKS, K), jnp.int32),
            pltpu.VMEM((CHUNKS, K), jnp.int32),
            pltpu.VMEM((K, DH), jnp.float32),
            pltpu.VMEM((K, DH), jnp.float32),
            pltpu.VMEM((K,), jnp.float32),
            pltpu.VMEM_SHARED((N_ACC, DH), jnp.float32),
            pltpu.VMEM_SHARED((N_ACC,), jnp.float32),
            pltpu.SemaphoreType.DMA,
            pltpu.SemaphoreType.DMA,
            pltpu.SemaphoreType.DMA,
            pltpu.SemaphoreType.DMA,
        ],
    )(xh, srcp, dstp, zrows, zcnt, ones)


def _tc_body(x_ref, agg0, agg1, cnt0, cnt1, batch_ref,
             wl0, wl1, bl, wr, aw, ab, lw, lb,
             out_ref, pooled_acc, gcnt_acc):
    j = pl.program_id(0)

    @pl.when(j == 0)
    def _init():
        pooled_acc[...] = jnp.zeros_like(pooled_acc)
        gcnt_acc[...] = jnp.zeros_like(gcnt_acc)

    inv = 1.0 / jnp.maximum(cnt0[...] + cnt1[...], 1.0)
    h = lax.dot_general(agg0[...] * inv, wl0[...], (((1,), (1,)), ((), ())),
                        preferred_element_type=jnp.float32)
    h = h + lax.dot_general(agg1[...] * inv, wl1[...],
                            (((1,), (1,)), ((), ())),
                            preferred_element_type=jnp.float32)
    h = h + lax.dot_general(x_ref[...], wr[...], (((1,), (1,)), ((), ())),
                            preferred_element_type=jnp.float32)
    h = jnp.maximum(h + bl[...], 0.0)
    att = jnp.sum(h * aw[...], axis=1, keepdims=True)
    att = jax.nn.sigmoid(att + ab[0, 0])
    hs = h * att
    onehot = (batch_ref[...] ==
              lax.broadcasted_iota(jnp.int32, (NBLK, G), 1)).astype(jnp.float32)
    pooled_acc[...] += lax.dot_general(onehot, hs, (((0,), (0,)), ((), ())),
                                       preferred_element_type=jnp.float32)
    gcnt_acc[...] += lax.dot_general(onehot, jnp.ones_like(hs),
                                     (((0,), (0,)), ((), ())),
                                     preferred_element_type=jnp.float32)

    @pl.when(j == NSTEPS - 1)
    def _final():
        pooled = pooled_acc[...] / jnp.maximum(gcnt_acc[...], 1.0)
        out_ref[...] = lax.dot_general(
            pooled, lw[...], (((1,), (1,)), ((), ())),
            preferred_element_type=jnp.float32) + lb[...]


@jax.jit
def _tc_dense(x, agg, cnt, batch, W_l, b_l, W_r, att_w, att_b, lin_w, lin_b):
    cnt3 = cnt.reshape(NC, N_ACC, 1)
    batch2 = batch.reshape(N, 1).astype(jnp.int32)
    const = lambda shape: pl.BlockSpec(shape, lambda j: tuple(0 for _ in shape))
    return pl.pallas_call(
        _tc_body,
        grid=(NSTEPS,),
        in_specs=[
            pl.BlockSpec((NBLK, D), lambda j: (j, 0)),
            pl.BlockSpec((NBLK, D), lambda j: (j, 0)),
            pl.BlockSpec((NBLK, D), lambda j: (j, 0)),
            pl.BlockSpec((NBLK, 1), lambda j: (j, 0)),
            pl.BlockSpec((NBLK, 1), lambda j: (j, 0)),
            pl.BlockSpec((NBLK, 1), lambda j: (j, 0)),
            const((D, D)),
            const((D, D)),
            const((1, D)),
            const((D, D)),
            const((1, D)),
            const((1, 1)),
            const((D, D)),
            const((1, D)),
        ],
        out_specs=pl.BlockSpec((G, D), lambda j: (0, 0)),
        out_shape=jax.ShapeDtypeStruct((G, D), jnp.float32),
        scratch_shapes=[
            pltpu.VMEM((G, D), jnp.float32),
            pltpu.VMEM((G, D), jnp.float32),
        ],
    )(x, agg[0], agg[1], cnt3[0], cnt3[1], batch2,
      W_l, W_l, b_l.reshape(1, D), W_r, att_w,
      att_b.reshape(1, 1), lin_w, lin_b.reshape(1, D))


def kernel(x, edge_index, batch, W_l, b_l, W_r, att_w, att_b, lin_w, lin_b):
    src = edge_index[0]
    dst = edge_index[1]
    pad = E_PAD - E
    srcp = jnp.concatenate(
        [src, jnp.zeros((pad,), src.dtype)]).reshape(NC * NS * CHUNKS, K)
    dstp = jnp.concatenate(
        [dst, jnp.full((pad,), DUMMY_DST, dst.dtype)]).reshape(NC * NS * CHUNKS, K)
    xh = x
    agg, cnt = _sc_aggregate(xh, srcp.astype(jnp.int32), dstp.astype(jnp.int32))
    return _tc_dense(x, agg, cnt, batch,
                     W_l, b_l, W_r, att_w, att_b, lin_w, lin_b)


# speedup vs baseline: 1.3522x; 1.3522x over previous
import jax
import jax.numpy as jnp
from jax import lax
from jax.experimental import pallas as pl
from jax.experimental.pallas import tpu as pltpu
from jax.experimental.pallas import tpu_sc as plsc

N = 10000
E = 320000
D = 128
G = 64

NC = 2
NS = 16
DH = D // NC

K = 256
CHUNKS = 80
W_EDGES = CHUNKS * K
E_PAD = NS * W_EDGES
N_ACC = 10240
ROWS_PER_TILE = N_ACC // NS
DUMMY_DST = N + 8

NBLK = 1000
NSTEPS = N // NBLK


def _sc_body(xh, srcp, dstp, zrows, zcnt, ones_hbm,
             agg_out, cnt_out,
             src_all, dst_all, rows_a, rows_b, ones_v, acc_sh, cnt_sh,
             gsem0, gsem1, ssem0, ssem1):
    c = lax.axis_index("c")
    s = lax.axis_index("s")
    rows = (rows_a, rows_b)
    gsem = (gsem0, gsem1)
    ssem = (ssem0, ssem1)

    pltpu.sync_copy(srcp.at[pl.ds(s * CHUNKS, CHUNKS)], src_all)
    pltpu.sync_copy(dstp.at[pl.ds(s * CHUNKS, CHUNKS)], dst_all)
    pltpu.sync_copy(ones_hbm, ones_v)

    base = s * ROWS_PER_TILE
    pltpu.sync_copy(zrows, acc_sh.at[pl.ds(base, ROWS_PER_TILE)])
    pltpu.sync_copy(zcnt, cnt_sh.at[pl.ds(base, ROWS_PER_TILE)])
    plsc.subcore_barrier()

    x_half = xh.at[c]

    def start_gather(j, b):
        pltpu.async_copy(x_half.at[src_all.at[j]], rows[b], gsem[b])

    def wait_gather(j, b):
        pltpu.make_async_copy(x_half.at[src_all.at[j]], rows[b],
                              gsem[b]).wait()

    def start_scatter(j, b):
        pltpu.async_copy(rows[b], acc_sh.at[dst_all.at[j]], ssem[b],
                         add=True)

    def wait_scatter(j, b):
        pltpu.make_async_copy(rows[b], acc_sh.at[dst_all.at[j]],
                              ssem[b]).wait()

    def count(j):
        pltpu.sync_copy(ones_v, cnt_sh.at[dst_all.at[j]], add=True)

    start_gather(0, 0)
    wait_gather(0, 0)
    start_scatter(0, 0)
    start_gather(1, 1)

    def pair(p, carry):
        for b in range(2):
            j = 2 * p + 1 + b
            wait_scatter(j - 1, b)
            start_gather(j + 1, b)
            wait_gather(j, 1 - b)
            start_scatter(j, 1 - b)

            cj = j - 1
            @pl.when(jnp.where(cj < CHUNKS // 2, c == 0, c == 1))
            def _():
                count(cj)
        return carry

    lax.fori_loop(0, CHUNKS // 2 - 1, pair, 0)
    j_last = CHUNKS - 1
    wait_scatter(j_last - 1, 0)
    wait_gather(j_last, 1)
    start_scatter(j_last, 1)
    wait_scatter(j_last, 1)

    @pl.when(c == 1)
    def _():
        count(j_last - 1)
        count(j_last)

    plsc.subcore_barrier()

    pltpu.sync_copy(acc_sh.at[pl.ds(base, ROWS_PER_TILE)],
                    agg_out.at[c].at[pl.ds(base, ROWS_PER_TILE)])

    pltpu.sync_copy(cnt_sh.at[pl.ds(base, ROWS_PER_TILE)],
                    cnt_out.at[c].at[pl.ds(base, ROWS_PER_TILE)])


@jax.jit
def _sc_aggregate(xh, srcp, dstp):
    zrows = jnp.zeros((ROWS_PER_TILE, DH), jnp.float32)
    zcnt = jnp.zeros((ROWS_PER_TILE,), jnp.float32)
    ones = jnp.ones((K,), jnp.float32)
    mesh = plsc.VectorSubcoreMesh(core_axis_name="c", subcore_axis_name="s")
    return pl.kernel(
        _sc_body,
        out_type=(
            jax.ShapeDtypeStruct((NC, N_ACC, DH), jnp.float32),
            jax.ShapeDtypeStruct((NC, N_ACC), jnp.float32),
        ),
        mesh=mesh,
        compiler_params=pltpu.CompilerParams(use_tc_tiling_on_sc=False),
        scratch_types=[
            pltpu.VMEM((CHUNKS, K), jnp.int32),
            pltpu.VMEM((CHUNKS, K), jnp.int32),
            pltpu.VMEM((K, DH), jnp.float32),
            pltpu.VMEM((K, DH), jnp.float32),
            pltpu.VMEM((K,), jnp.float32),
            pltpu.VMEM_SHARED((N_ACC, DH), jnp.float32),
            pltpu.VMEM_SHARED((N_ACC,), jnp.float32),
            pltpu.SemaphoreType.DMA,
            pltpu.SemaphoreType.DMA,
            pltpu.SemaphoreType.DMA,
            pltpu.SemaphoreType.DMA,
        ],
    )(xh, srcp, dstp, zrows, zcnt, ones)


def _tc_body(x_ref, agg0, agg1, cnt0, cnt1, batch_ref,
             wl0, wl1, bl, wr, aw, ab, lw, lb,
             out_ref, pooled_acc, gcnt_acc):
    j = pl.program_id(0)

    @pl.when(j == 0)
    def _init():
        pooled_acc[...] = jnp.zeros_like(pooled_acc)
        gcnt_acc[...] = jnp.zeros_like(gcnt_acc)

    inv = 1.0 / jnp.maximum(cnt0[...] + cnt1[...], 1.0)
    h = lax.dot_general(agg0[...] * inv, wl0[...], (((1,), (1,)), ((), ())),
                        preferred_element_type=jnp.float32)
    h = h + lax.dot_general(agg1[...] * inv, wl1[...],
                            (((1,), (1,)), ((), ())),
                            preferred_element_type=jnp.float32)
    h = h + lax.dot_general(x_ref[...], wr[...], (((1,), (1,)), ((), ())),
                            preferred_element_type=jnp.float32)
    h = jnp.maximum(h + bl[...], 0.0)
    att = jnp.sum(h * aw[...], axis=1, keepdims=True)
    att = jax.nn.sigmoid(att + ab[0, 0])
    hs = h * att
    onehot = (batch_ref[...] ==
              lax.broadcasted_iota(jnp.int32, (NBLK, G), 1)).astype(jnp.float32)
    pooled_acc[...] += lax.dot_general(onehot, hs, (((0,), (0,)), ((), ())),
                                       preferred_element_type=jnp.float32)
    gcnt_acc[...] += lax.dot_general(onehot, jnp.ones_like(hs),
                                     (((0,), (0,)), ((), ())),
                                     preferred_element_type=jnp.float32)

    @pl.when(j == NSTEPS - 1)
    def _final():
        pooled = pooled_acc[...] / jnp.maximum(gcnt_acc[...], 1.0)
        out_ref[...] = lax.dot_general(
            pooled, lw[...], (((1,), (1,)), ((), ())),
            preferred_element_type=jnp.float32) + lb[...]


@jax.jit
def _tc_dense(x, agg, cnt, batch, W_l, b_l, W_r, att_w, att_b, lin_w, lin_b):
    cnt3 = cnt.reshape(NC, N_ACC, 1)
    batch2 = batch.reshape(N, 1).astype(jnp.int32)
    const = lambda shape: pl.BlockSpec(shape, lambda j: tuple(0 for _ in shape))
    return pl.pallas_call(
        _tc_body,
        grid=(NSTEPS,),
        in_specs=[
            pl.BlockSpec((NBLK, D), lambda j: (j, 0)),
            pl.BlockSpec((NBLK, DH), lambda j: (j, 0)),
            pl.BlockSpec((NBLK, DH), lambda j: (j, 0)),
            pl.BlockSpec((NBLK, 1), lambda j: (j, 0)),
            pl.BlockSpec((NBLK, 1), lambda j: (j, 0)),
            pl.BlockSpec((NBLK, 1), lambda j: (j, 0)),
            const((D, DH)),
            const((D, DH)),
            const((1, D)),
            const((D, D)),
            const((1, D)),
            const((1, 1)),
            const((D, D)),
            const((1, D)),
        ],
        out_specs=pl.BlockSpec((G, D), lambda j: (0, 0)),
        out_shape=jax.ShapeDtypeStruct((G, D), jnp.float32),
        scratch_shapes=[
            pltpu.VMEM((G, D), jnp.float32),
            pltpu.VMEM((G, D), jnp.float32),
        ],
    )(x, agg[0], agg[1], cnt3[0], cnt3[1], batch2,
      W_l[:, :DH], W_l[:, DH:], b_l.reshape(1, D), W_r, att_w,
      att_b.reshape(1, 1), lin_w, lin_b.reshape(1, D))


def kernel(x, edge_index, batch, W_l, b_l, W_r, att_w, att_b, lin_w, lin_b):
    src = edge_index[0]
    dst = edge_index[1]
    pad = E_PAD - E
    srcp = jnp.concatenate(
        [src, jnp.zeros((pad,), src.dtype)]).reshape(NS * CHUNKS, K)
    dstp = jnp.concatenate(
        [dst, jnp.full((pad,), DUMMY_DST, dst.dtype)]).reshape(NS * CHUNKS, K)
    xh = jnp.stack([x[:, :DH], x[:, DH:]])
    agg, cnt = _sc_aggregate(xh, srcp.astype(jnp.int32), dstp.astype(jnp.int32))
    return _tc_dense(x, agg, cnt, batch,
                     W_l, b_l, W_r, att_w, att_b, lin_w, lin_b)


# speedup vs baseline: 1.3538x; 1.0012x over previous
import jax
import jax.numpy as jnp
from jax import lax
from jax.experimental import pallas as pl
from jax.experimental.pallas import tpu as pltpu
from jax.experimental.pallas import tpu_sc as plsc

N = 10000
E = 320000
D = 128
G = 64

NC = 2
NS = 16
DH = D // NC

K = 256
CHUNKS = 80
W_EDGES = CHUNKS * K
E_PAD = NS * W_EDGES
N_ACC = 10240
ROWS_PER_TILE = N_ACC // NS
DUMMY_DST = N + 8

NBLK = N
NSTEPS = N // NBLK


def _sc_body(xh, srcp, dstp, zrows, zcnt, ones_hbm,
             agg_out, cnt_out,
             src_all, dst_all, rows_a, rows_b, ones_v, acc_sh, cnt_sh,
             gsem0, gsem1, ssem0, ssem1):
    c = lax.axis_index("c")
    s = lax.axis_index("s")
    rows = (rows_a, rows_b)
    gsem = (gsem0, gsem1)
    ssem = (ssem0, ssem1)

    pltpu.sync_copy(srcp.at[pl.ds(s * CHUNKS, CHUNKS)], src_all)
    pltpu.sync_copy(dstp.at[pl.ds(s * CHUNKS, CHUNKS)], dst_all)
    pltpu.sync_copy(ones_hbm, ones_v)

    base = s * ROWS_PER_TILE
    pltpu.sync_copy(zrows, acc_sh.at[pl.ds(base, ROWS_PER_TILE)])
    pltpu.sync_copy(zcnt, cnt_sh.at[pl.ds(base, ROWS_PER_TILE)])
    plsc.subcore_barrier()

    x_half = xh.at[c]

    def start_gather(j, b):
        pltpu.async_copy(x_half.at[src_all.at[j]], rows[b], gsem[b])

    def wait_gather(j, b):
        pltpu.make_async_copy(x_half.at[src_all.at[j]], rows[b],
                              gsem[b]).wait()

    def start_scatter(j, b):
        pltpu.async_copy(rows[b], acc_sh.at[dst_all.at[j]], ssem[b],
                         add=True)

    def wait_scatter(j, b):
        pltpu.make_async_copy(rows[b], acc_sh.at[dst_all.at[j]],
                              ssem[b]).wait()

    def count(j):
        pltpu.sync_copy(ones_v, cnt_sh.at[dst_all.at[j]], add=True)

    start_gather(0, 0)
    wait_gather(0, 0)
    start_scatter(0, 0)
    start_gather(1, 1)

    def pair(p, carry):
        for b in range(2):
            j = 2 * p + 1 + b
            wait_scatter(j - 1, b)
            start_gather(j + 1, b)
            wait_gather(j, 1 - b)
            start_scatter(j, 1 - b)

            cj = j - 1
            @pl.when(jnp.where(cj < CHUNKS // 2, c == 0, c == 1))
            def _():
                count(cj)
        return carry

    lax.fori_loop(0, CHUNKS // 2 - 1, pair, 0)
    j_last = CHUNKS - 1
    wait_scatter(j_last - 1, 0)
    wait_gather(j_last, 1)
    start_scatter(j_last, 1)
    wait_scatter(j_last, 1)

    @pl.when(c == 1)
    def _():
        count(j_last - 1)
        count(j_last)

    plsc.subcore_barrier()

    pltpu.sync_copy(acc_sh.at[pl.ds(base, ROWS_PER_TILE)],
                    agg_out.at[c].at[pl.ds(base, ROWS_PER_TILE)])

    pltpu.sync_copy(cnt_sh.at[pl.ds(base, ROWS_PER_TILE)],
                    cnt_out.at[c].at[pl.ds(base, ROWS_PER_TILE)])


@jax.jit
def _sc_aggregate(xh, srcp, dstp):
    zrows = jnp.zeros((ROWS_PER_TILE, DH), jnp.float32)
    zcnt = jnp.zeros((ROWS_PER_TILE,), jnp.float32)
    ones = jnp.ones((K,), jnp.float32)
    mesh = plsc.VectorSubcoreMesh(core_axis_name="c", subcore_axis_name="s")
    return pl.kernel(
        _sc_body,
        out_type=(
            jax.ShapeDtypeStruct((NC, N_ACC, DH), jnp.float32),
            jax.ShapeDtypeStruct((NC, N_ACC), jnp.float32),
        ),
        mesh=mesh,
        compiler_params=pltpu.CompilerParams(use_tc_tiling_on_sc=False),
        scratch_types=[
            pltpu.VMEM((CHUNKS, K), jnp.int32),
            pltpu.VMEM((CHUNKS, K), jnp.int32),
            pltpu.VMEM((K, DH), jnp.float32),
            pltpu.VMEM((K, DH), jnp.float32),
            pltpu.VMEM((K,), jnp.float32),
            pltpu.VMEM_SHARED((N_ACC, DH), jnp.float32),
            pltpu.VMEM_SHARED((N_ACC,), jnp.float32),
            pltpu.SemaphoreType.DMA,
            pltpu.SemaphoreType.DMA,
            pltpu.SemaphoreType.DMA,
            pltpu.SemaphoreType.DMA,
        ],
    )(xh, srcp, dstp, zrows, zcnt, ones)


def _tc_body(x_ref, agg0, agg1, cnt0, cnt1, batch_ref,
             wl0, wl1, bl, wr, aw, ab, lw, lb,
             out_ref, pooled_acc, gcnt_acc):
    j = pl.program_id(0)

    @pl.when(j == 0)
    def _init():
        pooled_acc[...] = jnp.zeros_like(pooled_acc)
        gcnt_acc[...] = jnp.zeros_like(gcnt_acc)

    inv = 1.0 / jnp.maximum(cnt0[...] + cnt1[...], 1.0)
    h = lax.dot_general(agg0[...] * inv, wl0[...], (((1,), (1,)), ((), ())),
                        preferred_element_type=jnp.float32)
    h = h + lax.dot_general(agg1[...] * inv, wl1[...],
                            (((1,), (1,)), ((), ())),
                            preferred_element_type=jnp.float32)
    h = h + lax.dot_general(x_ref[...], wr[...], (((1,), (1,)), ((), ())),
                            preferred_element_type=jnp.float32)
    h = jnp.maximum(h + bl[...], 0.0)
    att = jnp.sum(h * aw[...], axis=1, keepdims=True)
    att = jax.nn.sigmoid(att + ab[0, 0])
    hs = h * att
    onehot = (batch_ref[...] ==
              lax.broadcasted_iota(jnp.int32, (NBLK, G), 1)).astype(jnp.float32)
    pooled_acc[...] += lax.dot_general(onehot, hs, (((0,), (0,)), ((), ())),
                                       preferred_element_type=jnp.float32)
    gcnt_acc[...] += lax.dot_general(onehot, jnp.ones_like(hs),
                                     (((0,), (0,)), ((), ())),
                                     preferred_element_type=jnp.float32)

    @pl.when(j == NSTEPS - 1)
    def _final():
        pooled = pooled_acc[...] / jnp.maximum(gcnt_acc[...], 1.0)
        out_ref[...] = lax.dot_general(
            pooled, lw[...], (((1,), (1,)), ((), ())),
            preferred_element_type=jnp.float32) + lb[...]


@jax.jit
def _tc_dense(x, agg, cnt, batch, W_l, b_l, W_r, att_w, att_b, lin_w, lin_b):
    cnt3 = cnt.reshape(NC, N_ACC, 1)
    batch2 = batch.reshape(N, 1).astype(jnp.int32)
    const = lambda shape: pl.BlockSpec(shape, lambda j: tuple(0 for _ in shape))
    return pl.pallas_call(
        _tc_body,
        grid=(NSTEPS,),
        in_specs=[
            pl.BlockSpec((NBLK, D), lambda j: (j, 0)),
            pl.BlockSpec((NBLK, DH), lambda j: (j, 0)),
            pl.BlockSpec((NBLK, DH), lambda j: (j, 0)),
            pl.BlockSpec((NBLK, 1), lambda j: (j, 0)),
            pl.BlockSpec((NBLK, 1), lambda j: (j, 0)),
            pl.BlockSpec((NBLK, 1), lambda j: (j, 0)),
            const((D, DH)),
            const((D, DH)),
            const((1, D)),
            const((D, D)),
            const((1, D)),
            const((1, 1)),
            const((D, D)),
            const((1, D)),
        ],
        out_specs=pl.BlockSpec((G, D), lambda j: (0, 0)),
        out_shape=jax.ShapeDtypeStruct((G, D), jnp.float32),
        scratch_shapes=[
            pltpu.VMEM((G, D), jnp.float32),
            pltpu.VMEM((G, D), jnp.float32),
        ],
    )(x, agg[0], agg[1], cnt3[0], cnt3[1], batch2,
      W_l[:, :DH], W_l[:, DH:], b_l.reshape(1, D), W_r, att_w,
      att_b.reshape(1, 1), lin_w, lin_b.reshape(1, D))


def kernel(x, edge_index, batch, W_l, b_l, W_r, att_w, att_b, lin_w, lin_b):
    src = edge_index[0]
    dst = edge_index[1]
    pad = E_PAD - E
    srcp = jnp.concatenate(
        [src, jnp.zeros((pad,), src.dtype)]).reshape(NS * CHUNKS, K)
    dstp = jnp.concatenate(
        [dst, jnp.full((pad,), DUMMY_DST, dst.dtype)]).reshape(NS * CHUNKS, K)
    xh = jnp.stack([x[:, :DH], x[:, DH:]])
    agg, cnt = _sc_aggregate(xh, srcp.astype(jnp.int32), dstp.astype(jnp.int32))
    return _tc_dense(x, agg, cnt, batch,
                     W_l, b_l, W_r, att_w, att_b, lin_w, lin_b)


# speedup vs baseline: 2.7427x; 2.0259x over previous
import jax
import jax.numpy as jnp
from jax import lax
from jax.experimental import pallas as pl
from jax.experimental.pallas import tpu as pltpu
from jax.experimental.pallas import tpu_sc as plsc

N = 10000
E = 320000
D = 128
G = 64

NC = 2
NS = 16
DH = D // NC

K = 256
CHUNKS = 80
W_EDGES = CHUNKS * K
E_PAD = NS * W_EDGES
N_ACC = 10240
ROWS_PER_TILE = N_ACC // NS
DUMMY_DST = N + 8

NBLK = N
NSTEPS = N // NBLK


def _sc_body(xh, srcp, dstp, zrows, zcnt, ones_hbm,
             agg_out, cnt_out,
             src_all, dst_all, rows_a, rows_b, ones_v, acc_sh, cnt_sh,
             gsem0, gsem1, ssem0, ssem1):
    c = lax.axis_index("c")
    s = lax.axis_index("s")
    rows = (rows_a, rows_b)
    gsem = (gsem0, gsem1)
    ssem = (ssem0, ssem1)

    pltpu.sync_copy(srcp.at[pl.ds(s * CHUNKS, CHUNKS)], src_all)
    pltpu.sync_copy(dstp.at[pl.ds(s * CHUNKS, CHUNKS)], dst_all)
    pltpu.sync_copy(ones_hbm, ones_v)

    base = s * ROWS_PER_TILE
    pltpu.sync_copy(zrows, acc_sh.at[pl.ds(base, ROWS_PER_TILE)])
    pltpu.sync_copy(zcnt, cnt_sh.at[pl.ds(base, ROWS_PER_TILE)])
    plsc.subcore_barrier()

    x_half = xh.at[c]

    def start_gather(j, b):
        pltpu.async_copy(x_half.at[src_all.at[j]], rows[b], gsem[b])

    def wait_gather(j, b):
        pltpu.make_async_copy(x_half.at[src_all.at[j]], rows[b],
                              gsem[b]).wait()

    def start_scatter(j, b):
        pltpu.async_copy(rows[b], acc_sh.at[dst_all.at[j]], ssem[b],
                         add=True)

    def wait_scatter(j, b):
        pltpu.make_async_copy(rows[b], acc_sh.at[dst_all.at[j]],
                              ssem[b]).wait()

    def count(j):
        pltpu.sync_copy(ones_v, cnt_sh.at[dst_all.at[j]], add=True)

    start_gather(0, 0)
    wait_gather(0, 0)
    start_scatter(0, 0)
    start_gather(1, 1)

    def pair(p, carry):
        for b in range(2):
            j = 2 * p + 1 + b
            wait_scatter(j - 1, b)
            start_gather(j + 1, b)
            wait_gather(j, 1 - b)
            start_scatter(j, 1 - b)

            cj = j - 1
            @pl.when(jnp.where(cj < CHUNKS // 2, c == 0, c == 1))
            def _():
                count(cj)
        return carry

    lax.fori_loop(0, CHUNKS // 2 - 1, pair, 0)
    j_last = CHUNKS - 1
    wait_scatter(j_last - 1, 0)
    wait_gather(j_last, 1)
    start_scatter(j_last, 1)
    wait_scatter(j_last, 1)

    @pl.when(c == 1)
    def _():
        count(j_last - 1)
        count(j_last)

    plsc.subcore_barrier()

    pltpu.sync_copy(acc_sh.at[pl.ds(base, ROWS_PER_TILE)],
                    agg_out.at[c].at[pl.ds(base, ROWS_PER_TILE)])

    pltpu.sync_copy(cnt_sh.at[pl.ds(base, ROWS_PER_TILE)],
                    cnt_out.at[c].at[pl.ds(base, ROWS_PER_TILE)])


@jax.jit
def _sc_aggregate(xh, srcp, dstp):
    zrows = jnp.zeros((ROWS_PER_TILE, DH), jnp.float32)
    zcnt = jnp.zeros((ROWS_PER_TILE,), jnp.float32)
    ones = jnp.ones((K,), jnp.float32)
    mesh = plsc.VectorSubcoreMesh(core_axis_name="c", subcore_axis_name="s")
    return pl.kernel(
        _sc_body,
        out_type=(
            jax.ShapeDtypeStruct((NC, N_ACC, DH), jnp.float32),
            jax.ShapeDtypeStruct((NC, N_ACC), jnp.float32),
        ),
        mesh=mesh,
        compiler_params=pltpu.CompilerParams(use_tc_tiling_on_sc=False),
        scratch_types=[
            pltpu.VMEM((CHUNKS, K), jnp.int32),
            pltpu.VMEM((CHUNKS, K), jnp.int32),
            pltpu.VMEM((K, DH), jnp.float32),
            pltpu.VMEM((K, DH), jnp.float32),
            pltpu.VMEM((K,), jnp.float32),
            pltpu.VMEM_SHARED((N_ACC, DH), jnp.float32),
            pltpu.VMEM_SHARED((N_ACC,), jnp.float32),
            pltpu.SemaphoreType.DMA,
            pltpu.SemaphoreType.DMA,
            pltpu.SemaphoreType.DMA,
            pltpu.SemaphoreType.DMA,
        ],
    )(xh, srcp, dstp, zrows, zcnt, ones)


def _tc_body(x_ref, agg0, agg1, cnt0, cnt1, batch_ref,
             wl0, wl1, bl, wr, aw, ab, lw, lb,
             out_ref, pooled_acc, gcnt_acc):
    j = pl.program_id(0)

    @pl.when(j == 0)
    def _init():
        pooled_acc[...] = jnp.zeros_like(pooled_acc)
        gcnt_acc[...] = jnp.zeros_like(gcnt_acc)

    inv = 1.0 / jnp.maximum(cnt0[...] + cnt1[...], 1.0)
    h = lax.dot_general(agg0[...] * inv, wl0[...], (((1,), (1,)), ((), ())),
                        preferred_element_type=jnp.float32)
    h = h + lax.dot_general(agg1[...] * inv, wl1[...],
                            (((1,), (1,)), ((), ())),
                            preferred_element_type=jnp.float32)
    h = h + lax.dot_general(x_ref[...], wr[...], (((1,), (1,)), ((), ())),
                            preferred_element_type=jnp.float32)
    h = jnp.maximum(h + bl[...], 0.0)
    att = jnp.sum(h * aw[...], axis=1, keepdims=True)
    att = jax.nn.sigmoid(att + ab[0, 0])
    hs = h * att
    onehot = (batch_ref[...] ==
              lax.broadcasted_iota(jnp.int32, (NBLK, G), 1)).astype(jnp.float32)
    pooled_acc[...] += lax.dot_general(onehot, hs, (((0,), (0,)), ((), ())),
                                       preferred_element_type=jnp.float32)
    gcnt_acc[...] += lax.dot_general(onehot, jnp.ones_like(hs),
                                     (((0,), (0,)), ((), ())),
                                     preferred_element_type=jnp.float32)

    @pl.when(j == NSTEPS - 1)
    def _final():
        pooled = pooled_acc[...] / jnp.maximum(gcnt_acc[...], 1.0)
        out_ref[...] = lax.dot_general(
            pooled, lw[...], (((1,), (1,)), ((), ())),
            preferred_element_type=jnp.float32) + lb[...]


@jax.jit
def _tc_dense(x, agg, cnt, batch, W_l, b_l, W_r, att_w, att_b, lin_w, lin_b):
    cnt3 = cnt.reshape(NC, N_ACC, 1)
    batch2 = batch.reshape(N, 1).astype(jnp.int32)
    const = lambda shape: pl.BlockSpec(shape, lambda j: tuple(0 for _ in shape))
    return pl.pallas_call(
        _tc_body,
        grid=(NSTEPS,),
        in_specs=[
            pl.BlockSpec((NBLK, D), lambda j: (j, 0)),
            pl.BlockSpec((NBLK, DH), lambda j: (j, 0)),
            pl.BlockSpec((NBLK, DH), lambda j: (j, 0)),
            pl.BlockSpec((NBLK, 1), lambda j: (j, 0)),
            pl.BlockSpec((NBLK, 1), lambda j: (j, 0)),
            pl.BlockSpec((NBLK, 1), lambda j: (j, 0)),
            const((D, DH)),
            const((D, DH)),
            const((1, D)),
            const((D, D)),
            const((1, D)),
            const((1, 1)),
            const((D, D)),
            const((1, D)),
        ],
        out_specs=pl.BlockSpec((G, D), lambda j: (0, 0)),
        out_shape=jax.ShapeDtypeStruct((G, D), jnp.float32),
        scratch_shapes=[
            pltpu.VMEM((G, D), jnp.float32),
            pltpu.VMEM((G, D), jnp.float32),
        ],
    )(x, agg[0], agg[1], cnt3[0], cnt3[1], batch2,
      W_l[:, :DH], W_l[:, DH:], b_l.reshape(1, D), W_r, att_w,
      att_b.reshape(1, 1), lin_w, lin_b.reshape(1, D))


def kernel(x, edge_index, batch, W_l, b_l, W_r, att_w, att_b, lin_w, lin_b):
    src = edge_index[0]
    dst = edge_index[1]
    pad = E_PAD - E
    pad_src = (jnp.arange(pad, dtype=src.dtype) * 37) % N
    pad_dst = N + 16 + (jnp.arange(pad, dtype=dst.dtype) % (N_ACC - N - 16))
    srcp = jnp.concatenate([src, pad_src]).reshape(NS * CHUNKS, K)
    dstp = jnp.concatenate([dst, pad_dst]).reshape(NS * CHUNKS, K)
    xh = jnp.stack([x[:, :DH], x[:, DH:]])
    agg, cnt = _sc_aggregate(xh, srcp.astype(jnp.int32), dstp.astype(jnp.int32))
    return _tc_dense(x, agg, cnt, batch,
                     W_l, b_l, W_r, att_w, att_b, lin_w, lin_b)
